# baseline (speedup 1.0000x reference)
import functools

import jax
import jax.numpy as jnp
from jax import lax
from jax.experimental import pallas as pl
from jax.experimental.pallas import tpu as pltpu
from jax.experimental.pallas import tpu_sc as plsc

B = 16384
V = 1000000
DIM = 32
NC = 2
NS = 16
NW = NC * NS
BPW = B // NW
CH = 128
NCH = BPW // CH
L = 16
GE = 4


def _body(user_h, item_h, ut_h, it_h, but_h, bit_h, gb_h, out_h,
          uidx, iidx, uwin0, uwin1, iwin0, iwin1, bu, bi, gbv, outv,
          sem, gsem0, gsem1):
    cid = lax.axis_index("c")
    sid = lax.axis_index("s")
    wid = sid * NC + cid
    base = wid * BPW

    cps = [
        pltpu.async_copy(user_h.at[pl.ds(base, BPW)], uidx, sem),
        pltpu.async_copy(item_h.at[pl.ds(base, BPW)], iidx, sem),
        pltpu.async_copy(gb_h, gbv, sem),
    ]
    for c in cps:
        c.wait()

    gs = []
    for j in range(NCH):
        gs.append(pltpu.async_copy(
            but_h.at[uidx.at[pl.ds(j * CH, CH)]], bu.at[pl.ds(j * CH, CH)],
            sem))
        gs.append(pltpu.async_copy(
            bit_h.at[iidx.at[pl.ds(j * CH, CH)]], bi.at[pl.ds(j * CH, CH)],
            sem))
    for g in gs:
        g.wait()

    gvec = gbv[...]
    lane_id = lax.iota(jnp.int32, L)
    eslot = lane_id & (GE - 1)
    ubufs = (uwin0, uwin1)
    ibufs = (iwin0, iwin1)
    gsems = (gsem0, gsem1)
    NPH = L // GE

    def group(g, carry):
        r0 = g * L
        u16 = uidx[pl.ds(r0, L)]
        i16 = iidx[pl.ds(r0, L)]
        ustart = (u16 // 128) * 128
        istart = (i16 // 128) * 128
        ulane = u16 - ustart
        ilane = i16 - istart

        def fetch(p):
            par = p & 1
            dcs = []
            for e in range(GE):
                su = pl.multiple_of(ustart[p * GE + e], 128)
                si = pl.multiple_of(istart[p * GE + e], 128)
                dcs.append(pltpu.async_copy(
                    ut_h.at[:, pl.ds(su, 128)], ubufs[par].at[e], gsems[par]))
                dcs.append(pltpu.async_copy(
                    it_h.at[:, pl.ds(si, 128)], ibufs[par].at[e], gsems[par]))
            return dcs

        def extract(p):
            par = p & 1
            acc = jnp.zeros((L,), jnp.float32)
            for d in range(DIM):
                dv = jnp.full((L,), d, jnp.int32)
                uv = plsc.load_gather(ubufs[par], [eslot, dv, ulane])
                iv = plsc.load_gather(ibufs[par], [eslot, dv, ilane])
                acc = acc + uv * iv
            return acc

        acc = jnp.zeros((L,), jnp.float32)
        pend = fetch(0)
        for p in range(NPH):
            nxt = fetch(p + 1) if p + 1 < NPH else []
            for c in pend:
                c.wait()
            pend = nxt
            acc_p = extract(p)
            acc = jnp.where((lane_id // GE) == p, acc_p, acc)
        res = acc + gvec + bu[pl.ds(r0, L)] + bi[pl.ds(r0, L)]
        outv[pl.ds(r0, L)] = jnp.clip(res, 1.0, 5.0)
        return carry

    lax.fori_loop(0, BPW // L, group, 0)
    pltpu.sync_copy(outv, out_h.at[pl.ds(base, BPW)])

_mesh = plsc.VectorSubcoreMesh(core_axis_name="c", subcore_axis_name="s")

_svd_sc = functools.partial(
    pl.kernel,
    mesh=_mesh,
    compiler_params=pltpu.CompilerParams(
        needs_layout_passes=False, use_tc_tiling_on_sc=True),
    out_type=jax.ShapeDtypeStruct((B,), jnp.float32),
    scratch_types=[
        pltpu.VMEM((BPW,), jnp.int32),
        pltpu.VMEM((BPW,), jnp.int32),
        pltpu.VMEM((GE, DIM, 128), jnp.float32),
        pltpu.VMEM((GE, DIM, 128), jnp.float32),
        pltpu.VMEM((GE, DIM, 128), jnp.float32),
        pltpu.VMEM((GE, DIM, 128), jnp.float32),
        pltpu.VMEM((BPW,), jnp.float32),
        pltpu.VMEM((BPW,), jnp.float32),
        pltpu.VMEM((L,), jnp.float32),
        pltpu.VMEM((BPW,), jnp.float32),
        pltpu.SemaphoreType.DMA,
        pltpu.SemaphoreType.DMA,
        pltpu.SemaphoreType.DMA,
    ],
)(_body)


@jax.jit
def kernel(user, item, user_table, item_table, bias_user_table,
           bias_item_table, global_bias):
    user = user.astype(jnp.int32)
    item = item.astype(jnp.int32)
    gb = jnp.full((L,), global_bias, jnp.float32)
    out = _svd_sc(user, item, user_table.T, item_table.T,
                  bias_user_table.reshape(-1), bias_item_table.reshape(-1),
                  gb)
    return out.reshape(1, B)


# speedup vs baseline: 1.0616x; 1.0616x over previous
import functools

import jax
import jax.numpy as jnp
from jax import lax
from jax.experimental import pallas as pl
from jax.experimental.pallas import tpu as pltpu
from jax.experimental.pallas import tpu_sc as plsc

B = 16384
V = 1000000
DIM = 32
NC = 2
NS = 16
NW = NC * NS
BPW = B // NW
CH = 128
NCH = BPW // CH
L = 16
GE = 8
NG = BPW // GE


def _body(user_h, item_h, ut_h, it_h, but_h, bit_h, gb_h, out_h,
          uidx, iidx, uwin, iwin, bu, bi, gbv, outv, sem, gsem):
    cid = lax.axis_index("c")
    sid = lax.axis_index("s")
    wid = sid * NC + cid
    base = wid * BPW

    cps = [
        pltpu.async_copy(user_h.at[pl.ds(base, BPW)], uidx, sem),
        pltpu.async_copy(item_h.at[pl.ds(base, BPW)], iidx, sem),
        pltpu.async_copy(gb_h, gbv, sem),
    ]
    for c in cps:
        c.wait()

    gs = []
    for j in range(NCH):
        gs.append(pltpu.async_copy(
            but_h.at[uidx.at[pl.ds(j * CH, CH)]], bu.at[pl.ds(j * CH, CH)],
            sem))
        gs.append(pltpu.async_copy(
            bit_h.at[iidx.at[pl.ds(j * CH, CH)]], bi.at[pl.ds(j * CH, CH)],
            sem))

    gvec = gbv[...]
    eslot = lax.iota(jnp.int32, L) & (GE - 1)
    lanehalf = lax.iota(jnp.int32, L) < GE

    def group(g, carry):
        r0 = g * L
        u16 = uidx[pl.ds(r0, L)]
        i16 = iidx[pl.ds(r0, L)]
        ustart = (u16 // 128) * 128
        istart = (i16 // 128) * 128
        ulane = u16 - ustart
        ilane = i16 - istart

        def fetch(e0):
            dcs = []
            for e in range(GE):
                su = pl.multiple_of(ustart[e0 + e], 128)
                si = pl.multiple_of(istart[e0 + e], 128)
                dcs.append(pltpu.async_copy(
                    ut_h.at[:, pl.ds(su, 128)], uwin.at[e], gsem))
                dcs.append(pltpu.async_copy(
                    it_h.at[:, pl.ds(si, 128)], iwin.at[e], gsem))
            for c in dcs:
                c.wait()

        def extract():
            acc = jnp.zeros((L,), jnp.float32)
            for d in range(DIM):
                dv = jnp.full((L,), d, jnp.int32)
                uv = plsc.load_gather(uwin, [eslot, dv, ulane])
                iv = plsc.load_gather(iwin, [eslot, dv, ilane])
                acc = acc + uv * iv
            return acc

        fetch(0)
        acc_a = extract()
        fetch(GE)
        acc_b = extract()
        acc = jnp.where(lanehalf, acc_a, acc_b)
        outv[pl.ds(r0, L)] = acc
        return carry

    lax.fori_loop(0, BPW // L, group, 0)
    for g in gs:
        g.wait()

    def finish(g, carry):
        r0 = g * L
        res = outv[pl.ds(r0, L)] + gvec + bu[pl.ds(r0, L)] + bi[pl.ds(r0, L)]
        outv[pl.ds(r0, L)] = jnp.clip(res, 1.0, 5.0)
        return carry

    lax.fori_loop(0, BPW // L, finish, 0)
    pltpu.sync_copy(outv, out_h.at[pl.ds(base, BPW)])

_mesh = plsc.VectorSubcoreMesh(core_axis_name="c", subcore_axis_name="s")

_svd_sc = functools.partial(
    pl.kernel,
    mesh=_mesh,
    compiler_params=pltpu.CompilerParams(
        needs_layout_passes=False, use_tc_tiling_on_sc=True),
    out_type=jax.ShapeDtypeStruct((B,), jnp.float32),
    scratch_types=[
        pltpu.VMEM((BPW,), jnp.int32),
        pltpu.VMEM((BPW,), jnp.int32),
        pltpu.VMEM((GE, DIM, 128), jnp.float32),
        pltpu.VMEM((GE, DIM, 128), jnp.float32),
        pltpu.VMEM((BPW,), jnp.float32),
        pltpu.VMEM((BPW,), jnp.float32),
        pltpu.VMEM((L,), jnp.float32),
        pltpu.VMEM((BPW,), jnp.float32),
        pltpu.SemaphoreType.DMA,
        pltpu.SemaphoreType.DMA,
    ],
)(_body)


@jax.jit
def kernel(user, item, user_table, item_table, bias_user_table,
           bias_item_table, global_bias):
    user = user.astype(jnp.int32)
    item = item.astype(jnp.int32)
    gb = jnp.full((L,), global_bias, jnp.float32)
    out = _svd_sc(user, item, user_table.T, item_table.T,
                  bias_user_table.reshape(-1), bias_item_table.reshape(-1),
                  gb)
    return out.reshape(1, B)
